# baseline (speedup 1.0000x reference)
import functools

import jax
import jax.numpy as jnp
from jax import lax
from jax.experimental import pallas as pl
from jax.experimental.pallas import tpu as pltpu
from jax.experimental.pallas import tpu_sc as plsc

EMB = 64
SCALE = 8.0

_info = plsc.get_sparse_core_info()
_NC = _info.num_cores
_NS = _info.num_subcores
_NW = _NC * _NS

_C = 128
_NBUF = 4


@functools.lru_cache(maxsize=None)
def _make_lookup(total: int, emb: int):
    per_w = total // _NW
    nchunk = per_w // _C
    mesh = plsc.VectorSubcoreMesh(core_axis_name="c", subcore_axis_name="s")

    @functools.partial(
        pl.kernel,
        out_type=jax.ShapeDtypeStruct((_NW, nchunk, _C, emb), jnp.float32),
        mesh=mesh,
        compiler_params=pltpu.CompilerParams(use_tc_tiling_on_sc=False),
        scratch_types=(
            [pltpu.VMEM((nchunk, _C), jnp.int32)]
            + [pltpu.VMEM((_C, emb), jnp.float32) for _ in range(2 * _NBUF)]
            + [pltpu.SemaphoreType.DMA for _ in range(2 * _NBUF)]
        ),
    )
    def lookup(tok_hbm, table_hbm, out_hbm, idx_v, *rest):
        in_bufs = rest[:_NBUF]
        out_bufs = rest[_NBUF:2 * _NBUF]
        gsems = rest[2 * _NBUF:3 * _NBUF]
        osems = rest[3 * _NBUF:4 * _NBUF]
        wid = lax.axis_index("c") * _NS + lax.axis_index("s")

        pltpu.sync_copy(tok_hbm.at[wid], idx_v)

        for b in range(_NBUF):
            pltpu.async_copy(table_hbm.at[idx_v.at[b]], in_bufs[b], gsems[b])

        @pl.loop(0, nchunk // _NBUF)
        def _outer(t):
            for b in range(_NBUF):
                g = t * _NBUF + b

                pltpu.make_async_copy(
                    table_hbm.at[idx_v.at[g]], in_bufs[b], gsems[b]
                ).wait()

                @pl.when(t > 0)
                def _():
                    pltpu.make_async_copy(
                        out_bufs[b], out_hbm.at[wid, g - _NBUF], osems[b]
                    ).wait()

                @pl.loop(0, _C, unroll=8)
                def _rows(r):
                    for u in range(emb // 16):
                        sl = pl.ds(u * 16, 16)
                        out_bufs[b][r, sl] = in_bufs[b][r, sl] * SCALE

                @pl.when(g + _NBUF < nchunk)
                def _():
                    pltpu.async_copy(
                        table_hbm.at[idx_v.at[g + _NBUF]], in_bufs[b], gsems[b]
                    )

                pltpu.async_copy(out_bufs[b], out_hbm.at[wid, g], osems[b])

        for b in range(_NBUF):
            g = nchunk - _NBUF + b
            pltpu.make_async_copy(
                out_bufs[b], out_hbm.at[wid, g], osems[b]
            ).wait()

    return lookup


def kernel(tokens, embedding_weight):
    total = tokens.shape[0] * tokens.shape[1]
    tok = tokens.astype(jnp.int32).reshape(_NW, total // _NW // _C, _C)
    out = _make_lookup(total, EMB)(tok, embedding_weight)
    return out.reshape(tokens.shape + (EMB,))
